# Initial kernel scaffold; baseline (speedup 1.0000x reference)
import jax
import jax.numpy as jnp
from jax.experimental import pallas as pl


def kernel(input_ids, token_type_ids, word_embeddings, token_type_embeddings, ln_gamma, ln_beta):
    raise NotImplementedError("write your pallas kernel here")


# speedup vs baseline: 2.3517x; 2.3517x over previous
import functools

import jax
import jax.numpy as jnp
from jax import lax
from jax.experimental import pallas as pl
from jax.experimental.pallas import tpu as pltpu
from jax.experimental.pallas import tpu_sc as plsc

_B = 4
_S = 4096
_D = 128
_EPS = 1e-12

_N = _B * _S
_NW = 32
_TOK_PER_W = _N // _NW
_CH = 128
_NCH = _TOK_PER_W // _CH

_ROWS_PER_BLK = 2048


def _sc_gather(idx2d, table):
    mesh = plsc.VectorSubcoreMesh(core_axis_name="c", subcore_axis_name="s")

    @functools.partial(
        pl.kernel,
        mesh=mesh,
        out_type=jax.ShapeDtypeStruct((_N, _D), jnp.float32),
        scratch_types=[
            pltpu.VMEM((_NCH, _CH), jnp.int32),
            pltpu.VMEM((_TOK_PER_W, _D), jnp.float32),
            pltpu.SemaphoreType.DMA,
        ],
    )
    def gather_kernel(idx_hbm, table_hbm, out_hbm, idx_v, rows_v, sem):
        wid = lax.axis_index("s") * 2 + lax.axis_index("c")
        pltpu.sync_copy(idx_hbm.at[pl.ds(wid * _NCH, _NCH)], idx_v)
        copies = [
            pltpu.async_copy(
                table_hbm.at[idx_v.at[j]],
                rows_v.at[pl.ds(j * _CH, _CH)],
                sem,
            )
            for j in range(_NCH)
        ]
        for c in copies:
            c.wait()
        pltpu.sync_copy(rows_v, out_hbm.at[pl.ds(wid * _TOK_PER_W, _TOK_PER_W)])

    return gather_kernel(idx2d, table)


def _tc_ln_body(x_ref, tt_ref, tte_ref, g_ref, b_ref, o_ref):
    x = x_ref[...]
    tt = tt_ref[...]
    tte = tte_ref[...]
    x = x + jnp.where(tt > 0, tte[1:2, :], tte[0:1, :])
    mean = jnp.mean(x, axis=-1, keepdims=True)
    xm = x - mean
    var = jnp.mean(xm * xm, axis=-1, keepdims=True)
    inv = lax.rsqrt(var + _EPS)
    o_ref[...] = xm * inv * g_ref[...] + b_ref[...]


def _tc_layernorm(x, tt, tte, gamma, beta, interpret=False):
    grid = (_N // _ROWS_PER_BLK,)
    return pl.pallas_call(
        _tc_ln_body,
        grid=grid,
        in_specs=[
            pl.BlockSpec((_ROWS_PER_BLK, _D), lambda i: (i, 0)),
            pl.BlockSpec((_ROWS_PER_BLK, 1), lambda i: (i, 0)),
            pl.BlockSpec((2, _D), lambda i: (0, 0)),
            pl.BlockSpec((1, _D), lambda i: (0, 0)),
            pl.BlockSpec((1, _D), lambda i: (0, 0)),
        ],
        out_specs=pl.BlockSpec((_ROWS_PER_BLK, _D), lambda i: (i, 0)),
        out_shape=jax.ShapeDtypeStruct((_N, _D), jnp.float32),
        interpret=interpret,
    )(x, tt, tte, gamma, beta)


def kernel(input_ids, token_type_ids, word_embeddings, token_type_embeddings,
           ln_gamma, ln_beta):
    idx2d = input_ids.reshape(_N // _CH, _CH)
    gathered = _sc_gather(idx2d, word_embeddings)
    tt = token_type_ids.reshape(_N, 1)
    out = _tc_layernorm(
        gathered, tt, token_type_embeddings,
        ln_gamma.reshape(1, _D), ln_beta.reshape(1, _D),
    )
    return out.reshape(_B, _S, _D)
